# baseline (speedup 1.0000x reference)
import functools

import jax
import jax.numpy as jnp
from jax import lax
from jax.experimental import pallas as pl
from jax.experimental.pallas import tpu as pltpu
from jax.experimental.pallas import tpu_sc as plsc

B, F, V, D = 4096, 26, 100000, 32
NW = 32
BPW = B // NW
H = D // 16


def _sc_gather_fm(idx_r, emb_flat, lin_flat):
    mesh = plsc.VectorSubcoreMesh(core_axis_name="c", subcore_axis_name="s")

    @functools.partial(
        pl.kernel,
        out_type=(
            jax.ShapeDtypeStruct((B,), jnp.float32),
            jax.ShapeDtypeStruct((B,), jnp.float32),
        ),
        mesh=mesh,
        scratch_types=[
            pltpu.VMEM((F, BPW), jnp.int32),
            pltpu.VMEM((F, BPW, D), jnp.float32),
            pltpu.VMEM((F, BPW), jnp.float32),
            pltpu.VMEM((BPW * 16,), jnp.float32),
            pltpu.VMEM((BPW,), jnp.float32),
            pltpu.VMEM((BPW,), jnp.float32),
            pltpu.SemaphoreType.DMA,
        ],
        compiler_params=pltpu.CompilerParams(
            needs_layout_passes=False, use_tc_tiling_on_sc=False),
    )
    def body(idx_hbm, emb_hbm, lin_hbm, eout_hbm, lout_hbm,
             idx_v, rows_v, linv_v, tbuf_v, eout_v, lout_v, sem):
        wid = lax.axis_index("c") * 16 + lax.axis_index("s")
        pltpu.sync_copy(idx_hbm.at[wid], idx_v)

        def fire(f, _):
            pltpu.async_copy(emb_hbm.at[idx_v.at[f]], rows_v.at[f], sem)
            pltpu.async_copy(lin_hbm.at[idx_v.at[f]], linv_v.at[f], sem)
            return 0
        lax.fori_loop(0, F, fire, 0)

        def drain(f, _):
            pltpu.make_async_copy(emb_hbm.at[idx_v.at[f]], rows_v.at[f], sem).wait()
            pltpu.make_async_copy(lin_hbm.at[idx_v.at[f]], linv_v.at[f], sem).wait()
            return 0
        lax.fori_loop(0, F, drain, 0)

        def ex_body(l, _):
            def f_body(f, carry):
                s0, s1, q0, q1 = carry
                v0 = rows_v[f, l, pl.ds(0, 16)]
                v1 = rows_v[f, l, pl.ds(16, 16)]
                return s0 + v0, s1 + v1, q0 + v0 * v0, q1 + v1 * v1
            z = jnp.zeros((16,), jnp.float32)
            s0, s1, q0, q1 = lax.fori_loop(0, F, f_body, (z, z, z, z))
            tbuf_v[pl.ds(l * 16, 16)] = s0 * s0 + s1 * s1 - q0 - q1
            return 0
        lax.fori_loop(0, BPW, ex_body, 0)

        idx16 = lax.iota(jnp.int32, 16)

        def eg_body(g, _):
            base = g * 256 + idx16 * 16
            def dd_body(dd, acc):
                return acc + plsc.load_gather(tbuf_v, [base + dd])
            acc = lax.fori_loop(0, 16, dd_body, jnp.zeros((16,), jnp.float32))
            eout_v[pl.ds(g * 16, 16)] = 0.5 * acc
            return 0
        lax.fori_loop(0, BPW // 16, eg_body, 0)

        def g_body(g, _):
            def f_body(f, acc):
                return acc + linv_v[f, pl.ds(g * 16, 16)]
            acc = lax.fori_loop(0, F, f_body, jnp.zeros((16,), jnp.float32))
            lout_v[pl.ds(g * 16, 16)] = acc
            return 0
        lax.fori_loop(0, BPW // 16, g_body, 0)

        pltpu.sync_copy(eout_v, eout_hbm.at[pl.ds(wid * BPW, BPW)])
        pltpu.sync_copy(lout_v, lout_hbm.at[pl.ds(wid * BPW, BPW)])

    return body(idx_r, emb_flat, lin_flat)


def _tc_broadcast(lin_col, emb_row, bias2):
    BR = 512

    def body(lin_ref, emb_ref, bias_ref, out_ref):
        out_ref[...] = lin_ref[...] + emb_ref[...] + bias_ref[0, 0]

    return pl.pallas_call(
        body,
        grid=(B // BR,),
        in_specs=[
            pl.BlockSpec((BR, 1), lambda i: (i, 0)),
            pl.BlockSpec((1, B), lambda i: (0, 0)),
            pl.BlockSpec(memory_space=pltpu.SMEM),
        ],
        out_specs=pl.BlockSpec((BR, B), lambda i: (i, 0)),
        out_shape=jax.ShapeDtypeStruct((B, B), jnp.float32),
    )(lin_col, emb_row, bias2)


def kernel(indices, emb_tables, lin_tables, bias):
    emb_flat = emb_tables.reshape(F * V, D)
    lin_flat = lin_tables.reshape(F * V)
    gidx = indices + (jnp.arange(F, dtype=jnp.int32) * V)[None, :]
    idx_r = gidx.reshape(NW, BPW, F).transpose(0, 2, 1)
    emb_logits, lin_sums = _sc_gather_fm(idx_r, emb_flat, lin_flat)
    out = _tc_broadcast(lin_sums.reshape(B, 1), emb_logits.reshape(1, B),
                        bias.reshape(1, 1))
    return out
